# baseline (speedup 1.0000x reference)
import functools

import jax
import jax.numpy as jnp
from jax import lax
from jax.experimental import pallas as pl
from jax.experimental.pallas import tpu as pltpu
from jax.experimental.pallas import tpu_sc as plsc

_NC = 2
_NS = 16
_B = 128
_G = 16
_H = 16
_EB = 32768


def _mesh():
    return plsc.VectorSubcoreMesh(core_axis_name="c", subcore_axis_name="s",
                                  num_cores=_NC, num_subcores=_NS)


def _pipeline(groups, fire_g, fire_s):
    gd = fire_g(groups[0])
    sd_old = []
    for i, g in enumerate(groups):
        for d in gd:
            d.wait()
        sd_new = fire_s(g)
        for d in sd_old:
            d.wait()
        if i + 1 < len(groups):
            gd = fire_g(groups[i + 1])
        sd_old = sd_new
    for d in sd_old:
        d.wait()


@functools.lru_cache(maxsize=None)
def _make_deg(nacc, arows, ngrp0, ngrp1):
    rt = nacc // _NS
    r0 = ngrp0 * _G

    def body(col_hbm, ones_hbm, zeros_hbm, out_hbm, colidx_v, ones_v, acc_sh,
             sem):
        cid = lax.axis_index("c")
        sid = lax.axis_index("s")
        base = jnp.where(cid == 0, sid * r0, _NS * r0 + sid * (ngrp1 * _G))
        pre = [pltpu.async_copy(col_hbm.at[pl.ds(base, r0)], colidx_v, sem),
               pltpu.async_copy(ones_hbm, ones_v, sem),
               pltpu.async_copy(zeros_hbm, acc_sh.at[pl.ds(sid * rt, rt)],
                                sem)]
        for d in pre:
            d.wait()
        plsc.subcore_barrier()

        def fire_s(g):
            return [pltpu.async_copy(ones_v, acc_sh.at[colidx_v.at[j]], sem,
                                     add=True)
                    for j in range(g * _G, (g + 1) * _G)]

        def no_g(g):
            return []

        _pipeline(list(range(ngrp1)), no_g, fire_s)
        if ngrp0 > ngrp1:
            @pl.when(cid == 0)
            def _():
                _pipeline(list(range(ngrp1, ngrp0)), no_g, fire_s)
        plsc.subcore_barrier()
        pltpu.sync_copy(acc_sh.at[pl.ds(sid * rt, rt)],
                        out_hbm.at[cid, pl.ds(sid * rt, rt)])

    return pl.kernel(
        body,
        out_type=jax.ShapeDtypeStruct((_NC, nacc, 1), jnp.float32),
        mesh=_mesh(),
        compiler_params=pltpu.CompilerParams(use_tc_tiling_on_sc=False),
        scratch_types=[
            pltpu.VMEM((ngrp0 * _G, _B), jnp.int32),
            pltpu.VMEM((_B, 1), jnp.float32),
            pltpu.VMEM_SHARED((nacc, 1), jnp.float32),
            pltpu.SemaphoreType.DMA,
        ],
    )


@functools.lru_cache(maxsize=None)
def _make_prop(nacc, arows, ngrp0, ngrp1):
    rt = nacc // _NS
    r0 = ngrp0 * _G

    def body(row_hbm, col_hbm, h_hbm, zeros_hbm, out_hbm,
             rowidx_v, colidx_v, rows_a, rows_b, h_sh, acc_sh, semg, sems):
        cid = lax.axis_index("c")
        sid = lax.axis_index("s")
        base = jnp.where(cid == 0, sid * r0, _NS * r0 + sid * (ngrp1 * _G))
        pre = [pltpu.async_copy(row_hbm.at[pl.ds(base, r0)], rowidx_v, semg),
               pltpu.async_copy(col_hbm.at[pl.ds(base, r0)], colidx_v, semg),
               pltpu.async_copy(h_hbm.at[pl.ds(sid * rt, rt)],
                                h_sh.at[pl.ds(sid * rt, rt)], semg),
               pltpu.async_copy(zeros_hbm, acc_sh.at[pl.ds(sid * rt, rt)],
                                semg)]
        for d in pre:
            d.wait()
        plsc.subcore_barrier()
        bufs = [rows_a, rows_b]

        def fire_g(g):
            buf = bufs[g % 2]
            return [
                pltpu.async_copy(h_sh.at[rowidx_v.at[g * _G + k]],
                                 buf.at[pl.ds(k * _B, _B)], semg)
                for k in range(_G)
            ]

        def fire_s(g):
            buf = bufs[g % 2]
            return [
                pltpu.async_copy(buf.at[pl.ds(k * _B, _B)],
                                 acc_sh.at[colidx_v.at[g * _G + k]], sems,
                                 add=True)
                for k in range(_G)
            ]

        _pipeline(list(range(ngrp1)), fire_g, fire_s)
        if ngrp0 > ngrp1:
            @pl.when(cid == 0)
            def _():
                _pipeline(list(range(ngrp1, ngrp0)), fire_g, fire_s)
        plsc.subcore_barrier()
        pltpu.sync_copy(acc_sh.at[pl.ds(sid * rt, rt)],
                        out_hbm.at[cid, pl.ds(sid * rt, rt)])

    return pl.kernel(
        body,
        out_type=jax.ShapeDtypeStruct((_NC, nacc, _H), jnp.float32),
        mesh=_mesh(),
        compiler_params=pltpu.CompilerParams(use_tc_tiling_on_sc=False),
        scratch_types=[
            pltpu.VMEM((ngrp0 * _G, _B), jnp.int32),
            pltpu.VMEM((ngrp0 * _G, _B), jnp.int32),
            pltpu.VMEM((_G * _B, _H), jnp.float32),
            pltpu.VMEM((_G * _B, _H), jnp.float32),
            pltpu.VMEM_SHARED((nacc, _H), jnp.float32),
            pltpu.VMEM_SHARED((nacc, _H), jnp.float32),
            pltpu.SemaphoreType.DMA,
            pltpu.SemaphoreType.DMA,
        ],
    )


def _dinvb(d_ref, ke_ref):
    d = d_ref[...]
    degb = d[0] + d[1]
    dinv8 = jnp.where(degb > 0, lax.rsqrt(jnp.maximum(degb, 1.0)), 0.0)
    return jnp.dot(dinv8, ke_ref[...], preferred_element_type=jnp.float32)


@functools.lru_cache(maxsize=None)
def _make_edgeprep(e, arows, n):
    grid = -(-e // _EB)

    def body(ei_ref, row_ref, col_ref):
        i = pl.program_id(0)
        r_io = lax.broadcasted_iota(jnp.int32, (_EB // _B, _B), 0)
        c_io = lax.broadcasted_iota(jnp.int32, (_EB // _B, _B), 1)
        eidx = i * _EB + r_io * _B + c_io
        valid = eidx < e
        ei = ei_ref[...]
        row_ref[...] = jnp.where(valid, ei[0].reshape(_EB // _B, _B), n)
        col_ref[...] = jnp.where(valid, ei[1].reshape(_EB // _B, _B), n)

    nblk = -(-arows // (_EB // _B))
    return pl.pallas_call(
        body,
        grid=(grid,),
        in_specs=[pl.BlockSpec((2, _EB), lambda i: (0, i))],
        out_specs=[pl.BlockSpec((_EB // _B, _B), lambda i: (i, 0)),
                   pl.BlockSpec((_EB // _B, _B), lambda i: (i, 0))],
        out_shape=(jax.ShapeDtypeStruct((nblk * (_EB // _B), _B), jnp.int32),
                   jax.ShapeDtypeStruct((nblk * (_EB // _B), _B), jnp.int32)),
    )


@functools.lru_cache(maxsize=None)
def _make_pre1mm(n, nacc, d_in):
    nr = n // 8
    rows = nacc // 8

    def body(x_ref, w_ref, v_ref, b_ref, h_ref, skip_ref):
        x = x_ref[...]
        h_ref[0:nr, :] = jnp.dot(x, w_ref[...],
                                 preferred_element_type=jnp.float32)
        h_ref[nr:, :] = jnp.zeros((rows - nr, _B), jnp.float32)
        s = (jnp.dot(x, v_ref[...], preferred_element_type=jnp.float32)
             + b_ref[0:1, :])
        skip_ref[0:nr, :] = s
        skip_ref[nr:, :] = jnp.zeros((rows - nr, _B), jnp.float32)

    return pl.pallas_call(
        body,
        out_shape=(jax.ShapeDtypeStruct((rows, _B), jnp.float32),
                   jax.ShapeDtypeStruct((rows, _B), jnp.float32)),
    )


@functools.lru_cache(maxsize=None)
def _make_scale(nacc):
    rows = nacc // 8

    def body(h_ref, d_ref, ke_ref, hp_ref):
        hp_ref[...] = _dinvb(d_ref, ke_ref) * h_ref[...]

    return pl.pallas_call(
        body,
        out_shape=jax.ShapeDtypeStruct((rows, _B), jnp.float32),
    )


@functools.lru_cache(maxsize=None)
def _make_mid(n, nacc):
    nr = n // 8
    rows = nacc // 8

    def body(a_ref, s_ref, d_ref, ke_ref, wb_ref, vb_ref, bb_ref,
             hp_ref, skip_ref):
        dinvb = _dinvb(d_ref, ke_ref)
        a = a_ref[...]
        aggb = a[0] + a[1]
        hprev = jnp.maximum(dinvb * aggb + s_ref[...], 0.0)
        h = jnp.dot(hprev, wb_ref[...], preferred_element_type=jnp.float32)
        hp = dinvb * h
        hp_ref[0:nr, :] = hp[:nr]
        hp_ref[nr:, :] = jnp.zeros((rows - nr, _B), jnp.float32)
        skip_ref[...] = (jnp.dot(hprev, vb_ref[...],
                                 preferred_element_type=jnp.float32)
                         + bb_ref[0:1, :])

    return pl.pallas_call(
        body,
        out_shape=(jax.ShapeDtypeStruct((rows, _B), jnp.float32),
                   jax.ShapeDtypeStruct((rows, _B), jnp.float32)),
    )


@functools.lru_cache(maxsize=None)
def _make_final(n, nacc, ncls):
    nr = n // 8

    def body(a_ref, s_ref, d_ref, ke_ref, kp_ref, km_ref, kr_ref, out_ref):
        dinvb = _dinvb(d_ref, ke_ref)
        a = a_ref[...]
        aggb = a[0] + a[1]
        zb = jnp.maximum(dinvb * aggb + s_ref[...], 0.0)[:nr]
        z10 = jnp.dot(zb, kp_ref[...], preferred_element_type=jnp.float32)
        ssum = jnp.dot(jnp.exp(zb), km_ref[...],
                       preferred_element_type=jnp.float32)
        lse = jnp.dot(jnp.log(ssum), kr_ref[...],
                      preferred_element_type=jnp.float32)
        out_ref[...] = z10 - lse

    return pl.pallas_call(
        body,
        out_shape=jax.ShapeDtypeStruct((nr, 8 * ncls), jnp.float32),
    )


def kernel(x, edge_index, W1, V1, b1, W2, V2, b2, W3, V3, b3):
    n, d_in = x.shape
    e = edge_index.shape[1]
    ncls = W3.shape[1]

    nrows = -(-e // _B)
    tot = -(-nrows // (_NS * _G))
    ngrp0 = max(1, min(tot - 1, round(0.6 * tot)))
    ngrp1 = tot - ngrp0
    ngrp0d = max(1, min(tot - 1, round(0.7 * tot)))
    ngrp1d = tot - ngrp0d
    margin = _G * max(ngrp0 - ngrp1, ngrp0d - ngrp1d)
    arows = _NS * _G * tot + margin
    nacc = -(-(n + 1) // (_NS * 8)) * (_NS * 8)
    rows = nacc // 8

    rowp, colp = _make_edgeprep(e, arows, n)(edge_index)

    rt = nacc // _NS
    zeros_rt = jnp.zeros((rt, _H), jnp.float32)
    zeros_rt1 = jnp.zeros((rt, 1), jnp.float32)
    ones_b1 = jnp.ones((_B, 1), jnp.float32)

    eye8 = jnp.eye(8, dtype=jnp.float32)

    def blk(w):
        wp = jnp.pad(w, ((0, 0), (0, _H - w.shape[1])))
        return jnp.kron(eye8, wp)

    def wideb(b):
        bp = jnp.concatenate([b, jnp.zeros((_H - b.shape[0],), b.dtype)])
        return jnp.broadcast_to(jnp.tile(bp, 8)[None, :], (8, _B))

    prop = _make_prop(nacc, rowp.shape[0], ngrp0, ngrp1)
    mid = _make_mid(n, nacc)

    x1024 = x.reshape(n // 8, 8 * d_in)
    W1k = jnp.kron(eye8, W1)
    V1k = jnp.kron(eye8, V1)

    sel = jnp.concatenate([jnp.eye(ncls, dtype=jnp.float32),
                           jnp.zeros((_H - ncls, ncls), jnp.float32)], axis=0)
    KE = jnp.kron(eye8, jnp.ones((1, _H), jnp.float32))
    KP = jnp.kron(eye8, sel)
    KM = jnp.kron(eye8, sel @ jnp.ones((ncls, 1), jnp.float32))
    KR = jnp.kron(eye8, jnp.ones((1, ncls), jnp.float32))

    h1raw, s1 = _make_pre1mm(n, nacc, d_in)(x1024, W1k, V1k, wideb(b1))
    deg = _make_deg(nacc, rowp.shape[0], ngrp0d, ngrp1d)(colp, ones_b1,
                                                         zeros_rt1)
    deg8 = deg.reshape(_NC, rows, 8)

    hp1 = _make_scale(nacc)(h1raw, deg8, KE)
    agg1 = prop(rowp, colp, hp1.reshape(nacc, _H), zeros_rt)
    hp2, s2 = mid(agg1.reshape(_NC, rows, _B), s1, deg8, KE,
                  blk(W2), blk(V2), wideb(b2))
    agg2 = prop(rowp, colp, hp2.reshape(nacc, _H), zeros_rt)
    hp3, s3 = mid(agg2.reshape(_NC, rows, _B), s2, deg8, KE,
                  blk(W3), blk(V3), wideb(b3))
    agg3 = prop(rowp, colp, hp3.reshape(nacc, _H), zeros_rt)
    out80 = _make_final(n, nacc, ncls)(agg3.reshape(_NC, rows, _B), s3,
                                       deg8, KE, KP, KM, KR)
    return out80.reshape(n, ncls)


# speedup vs baseline: 1.1058x; 1.1058x over previous
import functools

import jax
import jax.numpy as jnp
from jax import lax
from jax.experimental import pallas as pl
from jax.experimental.pallas import tpu as pltpu
from jax.experimental.pallas import tpu_sc as plsc

_NC = 2
_NS = 16
_B = 128
_G = 16
_H = 16
_EB = 32768


def _mesh():
    return plsc.VectorSubcoreMesh(core_axis_name="c", subcore_axis_name="s",
                                  num_cores=_NC, num_subcores=_NS)


def _pipeline(groups, fire_g, fire_s):
    gd = fire_g(groups[0])
    sd_old = []
    for i, g in enumerate(groups):
        for d in gd:
            d.wait()
        sd_new = fire_s(g)
        for d in sd_old:
            d.wait()
        if i + 1 < len(groups):
            gd = fire_g(groups[i + 1])
        sd_old = sd_new
    for d in sd_old:
        d.wait()


@functools.lru_cache(maxsize=None)
def _make_deg(nacc, arows, ngrp0, ngrp1):
    rt = nacc // _NS
    r0 = ngrp0 * _G

    def body(col_hbm, ones_hbm, zeros_hbm, out_hbm, colidx_v, ones_v, acc_sh,
             sem):
        cid = lax.axis_index("c")
        sid = lax.axis_index("s")
        base = jnp.where(cid == 0, sid * r0, _NS * r0 + sid * (ngrp1 * _G))
        pre = [pltpu.async_copy(col_hbm.at[pl.ds(base, r0)], colidx_v, sem),
               pltpu.async_copy(ones_hbm, ones_v, sem),
               pltpu.async_copy(zeros_hbm, acc_sh.at[pl.ds(sid * rt, rt)],
                                sem)]
        for d in pre:
            d.wait()
        plsc.subcore_barrier()

        def fire_s(g):
            return [pltpu.async_copy(ones_v, acc_sh.at[colidx_v.at[j]], sem,
                                     add=True)
                    for j in range(g * _G, (g + 1) * _G)]

        def no_g(g):
            return []

        _pipeline(list(range(ngrp1)), no_g, fire_s)
        if ngrp0 > ngrp1:
            @pl.when(cid == 0)
            def _():
                _pipeline(list(range(ngrp1, ngrp0)), no_g, fire_s)
        plsc.subcore_barrier()
        pltpu.sync_copy(acc_sh.at[pl.ds(sid * rt, rt)],
                        out_hbm.at[cid, pl.ds(sid * rt, rt)])

    return pl.kernel(
        body,
        out_type=jax.ShapeDtypeStruct((_NC, nacc, _H), jnp.float32),
        mesh=_mesh(),
        compiler_params=pltpu.CompilerParams(use_tc_tiling_on_sc=False),
        scratch_types=[
            pltpu.VMEM((ngrp0 * _G, _B), jnp.int32),
            pltpu.VMEM((_B, _H), jnp.float32),
            pltpu.VMEM_SHARED((nacc, _H), jnp.float32),
            pltpu.SemaphoreType.DMA,
        ],
    )


@functools.lru_cache(maxsize=None)
def _make_prop(nacc, arows, ngrp0, ngrp1):
    rt = nacc // _NS
    r0 = ngrp0 * _G

    def body(row_hbm, col_hbm, h_hbm, zeros_hbm, out_hbm,
             rowidx_v, colidx_v, rows_a, rows_b, h_sh, acc_sh, semg, sems):
        cid = lax.axis_index("c")
        sid = lax.axis_index("s")
        base = jnp.where(cid == 0, sid * r0, _NS * r0 + sid * (ngrp1 * _G))
        pre = [pltpu.async_copy(row_hbm.at[pl.ds(base, r0)], rowidx_v, semg),
               pltpu.async_copy(col_hbm.at[pl.ds(base, r0)], colidx_v, semg),
               pltpu.async_copy(h_hbm.at[pl.ds(sid * rt, rt)],
                                h_sh.at[pl.ds(sid * rt, rt)], semg),
               pltpu.async_copy(zeros_hbm, acc_sh.at[pl.ds(sid * rt, rt)],
                                semg)]
        for d in pre:
            d.wait()
        plsc.subcore_barrier()
        bufs = [rows_a, rows_b]

        def fire_g(g):
            buf = bufs[g % 2]
            return [
                pltpu.async_copy(h_sh.at[rowidx_v.at[g * _G + k]],
                                 buf.at[pl.ds(k * _B, _B)], semg)
                for k in range(_G)
            ]

        def fire_s(g):
            buf = bufs[g % 2]
            return [
                pltpu.async_copy(buf.at[pl.ds(k * _B, _B)],
                                 acc_sh.at[colidx_v.at[g * _G + k]], sems,
                                 add=True)
                for k in range(_G)
            ]

        _pipeline(list(range(ngrp1)), fire_g, fire_s)
        if ngrp0 > ngrp1:
            @pl.when(cid == 0)
            def _():
                _pipeline(list(range(ngrp1, ngrp0)), fire_g, fire_s)
        plsc.subcore_barrier()
        pltpu.sync_copy(acc_sh.at[pl.ds(sid * rt, rt)],
                        out_hbm.at[cid, pl.ds(sid * rt, rt)])

    return pl.kernel(
        body,
        out_type=jax.ShapeDtypeStruct((_NC, nacc, _H), jnp.float32),
        mesh=_mesh(),
        compiler_params=pltpu.CompilerParams(use_tc_tiling_on_sc=False),
        scratch_types=[
            pltpu.VMEM((ngrp0 * _G, _B), jnp.int32),
            pltpu.VMEM((ngrp0 * _G, _B), jnp.int32),
            pltpu.VMEM((_G * _B, _H), jnp.float32),
            pltpu.VMEM((_G * _B, _H), jnp.float32),
            pltpu.VMEM_SHARED((nacc, _H), jnp.float32),
            pltpu.VMEM_SHARED((nacc, _H), jnp.float32),
            pltpu.SemaphoreType.DMA,
            pltpu.SemaphoreType.DMA,
        ],
    )


def _dinvb(d_ref):
    d = d_ref[...]
    degb = d[0] + d[1]
    return jnp.where(degb > 0, lax.rsqrt(jnp.maximum(degb, 1.0)), 0.0)


@functools.lru_cache(maxsize=None)
def _make_edgeprep(e, arows, n):
    grid = -(-e // _EB)

    def body(ei_ref, row_ref, col_ref):
        i = pl.program_id(0)
        r_io = lax.broadcasted_iota(jnp.int32, (_EB // _B, _B), 0)
        c_io = lax.broadcasted_iota(jnp.int32, (_EB // _B, _B), 1)
        eidx = i * _EB + r_io * _B + c_io
        valid = eidx < e
        ei = ei_ref[...]
        row_ref[...] = jnp.where(valid, ei[0].reshape(_EB // _B, _B), n)
        col_ref[...] = jnp.where(valid, ei[1].reshape(_EB // _B, _B), n)

    nblk = -(-arows // (_EB // _B))
    return pl.pallas_call(
        body,
        grid=(grid,),
        in_specs=[pl.BlockSpec((2, _EB), lambda i: (0, i))],
        out_specs=[pl.BlockSpec((_EB // _B, _B), lambda i: (i, 0)),
                   pl.BlockSpec((_EB // _B, _B), lambda i: (i, 0))],
        out_shape=(jax.ShapeDtypeStruct((nblk * (_EB // _B), _B), jnp.int32),
                   jax.ShapeDtypeStruct((nblk * (_EB // _B), _B), jnp.int32)),
    )


@functools.lru_cache(maxsize=None)
def _make_pre1mm(n, nacc, d_in):
    nr = n // 8
    rows = nacc // 8

    def body(x_ref, w_ref, v_ref, b_ref, h_ref, skip_ref):
        x = x_ref[...]
        h_ref[0:nr, :] = jnp.dot(x, w_ref[...],
                                 preferred_element_type=jnp.float32)
        h_ref[nr:, :] = jnp.zeros((rows - nr, _B), jnp.float32)
        s = (jnp.dot(x, v_ref[...], preferred_element_type=jnp.float32)
             + b_ref[0:1, :])
        skip_ref[0:nr, :] = s
        skip_ref[nr:, :] = jnp.zeros((rows - nr, _B), jnp.float32)

    return pl.pallas_call(
        body,
        out_shape=(jax.ShapeDtypeStruct((rows, _B), jnp.float32),
                   jax.ShapeDtypeStruct((rows, _B), jnp.float32)),
    )


@functools.lru_cache(maxsize=None)
def _make_scale(nacc):
    rows = nacc // 8

    def body(h_ref, d_ref, hp_ref):
        hp_ref[...] = _dinvb(d_ref) * h_ref[...]

    return pl.pallas_call(
        body,
        out_shape=jax.ShapeDtypeStruct((rows, _B), jnp.float32),
    )


@functools.lru_cache(maxsize=None)
def _make_mid(n, nacc):
    nr = n // 8
    rows = nacc // 8

    def body(a_ref, s_ref, d_ref, wb_ref, vb_ref, bb_ref,
             hp_ref, skip_ref):
        dinvb = _dinvb(d_ref)
        a = a_ref[...]
        aggb = a[0] + a[1]
        hprev = jnp.maximum(dinvb * aggb + s_ref[...], 0.0)
        h = jnp.dot(hprev, wb_ref[...], preferred_element_type=jnp.float32)
        hp = dinvb * h
        hp_ref[0:nr, :] = hp[:nr]
        hp_ref[nr:, :] = jnp.zeros((rows - nr, _B), jnp.float32)
        skip_ref[...] = (jnp.dot(hprev, vb_ref[...],
                                 preferred_element_type=jnp.float32)
                         + bb_ref[0:1, :])

    return pl.pallas_call(
        body,
        out_shape=(jax.ShapeDtypeStruct((rows, _B), jnp.float32),
                   jax.ShapeDtypeStruct((rows, _B), jnp.float32)),
    )


@functools.lru_cache(maxsize=None)
def _make_final(n, nacc, ncls):
    nr = n // 8

    def body(a_ref, s_ref, d_ref, kp_ref, km_ref, kr_ref, out_ref):
        dinvb = _dinvb(d_ref)
        a = a_ref[...]
        aggb = a[0] + a[1]
        zb = jnp.maximum(dinvb * aggb + s_ref[...], 0.0)[:nr]
        z10 = jnp.dot(zb, kp_ref[...], preferred_element_type=jnp.float32)
        ssum = jnp.dot(jnp.exp(zb), km_ref[...],
                       preferred_element_type=jnp.float32)
        lse = jnp.dot(jnp.log(ssum), kr_ref[...],
                      preferred_element_type=jnp.float32)
        out_ref[...] = z10 - lse

    return pl.pallas_call(
        body,
        out_shape=jax.ShapeDtypeStruct((nr, 8 * ncls), jnp.float32),
    )


def kernel(x, edge_index, W1, V1, b1, W2, V2, b2, W3, V3, b3):
    n, d_in = x.shape
    e = edge_index.shape[1]
    ncls = W3.shape[1]

    nrows = -(-e // _B)
    tot = -(-nrows // (_NS * _G))
    ngrp0 = max(1, min(tot - 1, round(0.6 * tot)))
    ngrp1 = tot - ngrp0
    ngrp0d = max(1, min(tot - 1, round(0.7 * tot)))
    ngrp1d = tot - ngrp0d
    margin = _G * max(ngrp0 - ngrp1, ngrp0d - ngrp1d)
    arows = _NS * _G * tot + margin
    nacc = -(-(n + 1) // (_NS * 8)) * (_NS * 8)
    rows = nacc // 8

    rowp, colp = _make_edgeprep(e, arows, n)(edge_index)

    rt = nacc // _NS
    zeros_rt = jnp.zeros((rt, _H), jnp.float32)
    ones_b = jnp.ones((_B, _H), jnp.float32)

    eye8 = jnp.eye(8, dtype=jnp.float32)

    def blk(w):
        wp = jnp.pad(w, ((0, 0), (0, _H - w.shape[1])))
        return jnp.kron(eye8, wp)

    def wideb(b):
        bp = jnp.concatenate([b, jnp.zeros((_H - b.shape[0],), b.dtype)])
        return jnp.broadcast_to(jnp.tile(bp, 8)[None, :], (8, _B))

    prop = _make_prop(nacc, rowp.shape[0], ngrp0, ngrp1)
    mid = _make_mid(n, nacc)

    x1024 = x.reshape(n // 8, 8 * d_in)
    W1k = jnp.kron(eye8, W1)
    V1k = jnp.kron(eye8, V1)

    sel = jnp.concatenate([jnp.eye(ncls, dtype=jnp.float32),
                           jnp.zeros((_H - ncls, ncls), jnp.float32)], axis=0)
    KP = jnp.kron(eye8, sel)
    KM = jnp.kron(eye8, sel @ jnp.ones((ncls, 1), jnp.float32))
    KR = jnp.kron(eye8, jnp.ones((1, ncls), jnp.float32))

    h1raw, s1 = _make_pre1mm(n, nacc, d_in)(x1024, W1k, V1k, wideb(b1))
    deg = _make_deg(nacc, rowp.shape[0], ngrp0d, ngrp1d)(colp, ones_b,
                                                         zeros_rt)
    deg128 = deg.reshape(_NC, rows, _B)

    hp1 = _make_scale(nacc)(h1raw, deg128)
    agg1 = prop(rowp, colp, hp1.reshape(nacc, _H), zeros_rt)
    hp2, s2 = mid(agg1.reshape(_NC, rows, _B), s1, deg128,
                  blk(W2), blk(V2), wideb(b2))
    agg2 = prop(rowp, colp, hp2.reshape(nacc, _H), zeros_rt)
    hp3, s3 = mid(agg2.reshape(_NC, rows, _B), s2, deg128,
                  blk(W3), blk(V3), wideb(b3))
    agg3 = prop(rowp, colp, hp3.reshape(nacc, _H), zeros_rt)
    out80 = _make_final(n, nacc, ncls)(agg3.reshape(_NC, rows, _B), s3,
                                       deg128, KP, KM, KR)
    return out80.reshape(n, ncls)
